# baseline (speedup 1.0000x reference)
import functools

import jax
import jax.numpy as jnp
from jax import lax
from jax.experimental import pallas as pl
from jax.experimental.pallas import tpu as pltpu
from jax.experimental.pallas import tpu_sc as plsc

B = 16
N = 20000
K = 500
M = N // 2
NV = M // 16
KPAD = 512
PAD_KEY = -(2 ** 31)
PAD_IDX = 0x7FFFFFFF
MASK31 = 0x7FFFFFFF


def _body(lgi_hbm, lgf_hbm, bx_hbm, sc_hbm,
          vals_hbm, boxes_hbm, cnt_hbm,
          keys_v, hist_v, tot_v, par_v,
          cand_k, cand_i, sk_v, sp_v,
          gbx_v, obx_v, vals_v, vov_v, gidx_v,
          stage_v, scale_v, cnt_v,
          sh_hist, sh_meta, sh_ck, sh_ci, sh_bx, sh_vals, sh_cnt,
          sem):
    c = lax.axis_index("c")
    s = lax.axis_index("s")
    slot = s // 2
    h = s % 2
    row = c * 8 + slot
    iota = lax.iota(jnp.int32, 16)
    ones16 = jnp.ones((16,), jnp.int32)
    zero16 = jnp.zeros((16,), jnp.int32)
    lane_off = iota

    pltpu.sync_copy(lgi_hbm.at[pl.ds(row * N + h * M, M)], keys_v)

    def zero_hist(i, _):
        hist_v[pl.ds(i * 16, 16)] = zero16
        return 0
    lax.fori_loop(0, 256, zero_hist, 0, unroll=8)

    def key_pass(i, cnt0):
        b = keys_v[pl.ds(i * 16, 16)]
        key = b ^ ((b >> 31) & MASK31)
        keys_v[pl.ds(i * 16, 16)] = key
        d0 = (key >> 24) + 128
        plsc.addupdate_scatter(hist_v, [d0 * 16 + lane_off], ones16)
        return cnt0 + jnp.sum((key > 0).astype(jnp.int32))
    with jax.named_scope("keypass"):
        cnt0 = lax.fori_loop(0, NV, key_pass, jnp.int32(0), unroll=5)

    q = jnp.int32(K)
    P = jnp.int32(0)
    D3 = jnp.int32(0)
    for lvl in range(4):
        if lvl > 0:
            lax.fori_loop(0, 256, zero_hist, 0, unroll=8)
            sh_amt = 24 - 8 * lvl

            def hist_pass(i, _, sh_amt=sh_amt, P=P):
                key = keys_v[pl.ds(i * 16, 16)]
                m = (key >> (sh_amt + 8)) == P
                d = (key >> sh_amt) & 0xFF
                plsc.addupdate_scatter(hist_v, [d * 16 + lane_off], ones16,
                                       mask=m)
                return 0
            with jax.named_scope("histpass"):
                lax.fori_loop(0, NV, hist_pass, 0, unroll=5)

        def lane_reduce(g, _):
            acc = zero16
            for j in range(16):
                v = hist_v[pl.ds((g * 16 + j) * 16, 16)]
                acc = jnp.where(iota == j, jnp.sum(v), acc)
            tot_v[pl.ds(g * 16, 16)] = acc
            return 0
        with jax.named_scope("lanereduce"):
            lax.fori_loop(0, 16, lane_reduce, 0, unroll=2)

        pltpu.sync_copy(tot_v, sh_hist.at[slot * 2 + h])
        plsc.subcore_barrier()
        pltpu.sync_copy(sh_hist.at[slot * 2 + (1 - h)], par_v)

        def combine(g, _):
            bse = g * 16
            par_v[pl.ds(bse, 16)] = (par_v[pl.ds(bse, 16)]
                                     + tot_v[pl.ds(bse, 16)])
            return 0
        lax.fori_loop(0, 16, combine, 0, unroll=4)

        def find_bin(j, carry, q=q):
            found, D, qn, running = carry
            g = 15 - j
            v = par_v[pl.ds(g * 16, 16)]
            ssum = jnp.sum(v)
            rev = lax.rev(v, (0,))
            cum = jnp.cumsum(rev)
            here = jnp.logical_and(jnp.logical_not(found),
                                   (running + ssum) >= q)
            crossed = (running + cum) >= q
            jstar = jnp.min(jnp.where(crossed, iota, 16))
            cum_j = jnp.sum(jnp.where(iota == jstar, cum, 0))
            rev_j = jnp.sum(jnp.where(iota == jstar, rev, 0))
            D_new = g * 16 + 15 - jstar
            q_new = q - (running + cum_j - rev_j)
            D = jnp.where(here, D_new, D)
            qn = jnp.where(here, q_new, qn)
            found = jnp.logical_or(found, here)
            return (found, D, qn, running + ssum)
        _, D, q, _ = lax.fori_loop(
            0, 16, find_bin,
            (jnp.bool_(False), jnp.int32(0), jnp.int32(0), jnp.int32(0)))
        if lvl == 0:
            P = D - 128
        else:
            P = P * 256 + D
        if lvl == 3:
            D3 = D

    T = P
    r = q
    pltpu.sync_copy(sh_hist.at[slot * 2], tot_v)
    d3base = D3 // 16 * 16
    e0vec = tot_v[pl.ds(d3base, 16)]
    e0 = jnp.sum(jnp.where(iota == D3 - d3base, e0vec, 0))
    r0 = jnp.minimum(r, e0)
    quota = jnp.where(h == 0, r0, r - r0)

    def prefill(i, _):
        cand_k[pl.ds(i * 16, 16)] = jnp.full((16,), PAD_KEY, jnp.int32)
        cand_i[pl.ds(i * 16, 16)] = jnp.full((16,), PAD_IDX, jnp.int32)
        return 0
    lax.fori_loop(0, KPAD // 16, prefill, 0, unroll=8)

    base_idx = h * M

    def compact(i, carry):
        nsel, eqrun = carry
        key = keys_v[pl.ds(i * 16, 16)]
        m_gt = key > T
        m_eq = key == T
        has = jnp.logical_or(jnp.any(m_gt), jnp.any(m_eq))

        def do_vreg():
            eq_c = jnp.cumsum(m_eq.astype(jnp.int32))
            take = jnp.logical_and(m_eq, (eqrun + eq_c) <= quota)
            m = jnp.logical_or(m_gt, take)
            mi = m.astype(jnp.int32)
            pos = jnp.cumsum(mi) - mi + nsel
            plsc.store_scatter(cand_k, [pos], key, mask=m)
            idxv = base_idx + i * 16 + iota
            plsc.store_scatter(cand_i, [pos], idxv, mask=m)
            return (nsel + jnp.sum(mi),
                    eqrun + jnp.sum(m_eq.astype(jnp.int32)))

        return lax.cond(has, do_vreg, lambda: (nsel, eqrun))
    with jax.named_scope("compact"):
        nsel, _ = lax.fori_loop(0, NV, compact, (jnp.int32(0), jnp.int32(0)), unroll=5)

    meta = jnp.where(iota == 0, nsel, jnp.where(iota == 1, cnt0, 0))
    stage_v[...] = meta
    pltpu.sync_copy(stage_v, sh_meta.at[slot * 2 + h])

    @pl.when(h == 0)
    def _():
        pltpu.sync_copy(cand_k, sh_ck.at[pl.ds(slot * 1024, KPAD)])
        pltpu.sync_copy(cand_i, sh_ci.at[pl.ds(slot * 1024, KPAD)])
    plsc.subcore_barrier()

    pltpu.sync_copy(sh_meta.at[slot * 2 + (1 - h)], stage_v)
    pmeta = stage_v[...]
    partner_nsel = pmeta[0]
    partner_cnt0 = pmeta[1]

    @pl.when(h == 1)
    def _():
        n0p = (partner_nsel + 7) // 8 * 8
        pltpu.sync_copy(cand_k, sh_ck.at[pl.ds(slot * 1024 + n0p, KPAD)])
        pltpu.sync_copy(cand_i, sh_ci.at[pl.ds(slot * 1024 + n0p, KPAD)])

    @pl.when(h == 0)
    def _():
        cvec = jnp.where(iota == 0, cnt0 + partner_cnt0, 0)
        cnt_v[...] = cvec
        pltpu.sync_copy(cnt_v, sh_cnt.at[slot])
    plsc.subcore_barrier()

    pltpu.sync_copy(sh_ck.at[pl.ds(slot * 1024, KPAD)], cand_k)
    pltpu.sync_copy(sh_ci.at[pl.ds(slot * 1024, KPAD)], cand_i)

    @pl.when(h == 1)
    def _():
        def mk_idx(g, _):
            ci = cand_i[pl.ds(g * 16, 16)]
            gi = jnp.minimum(ci, N - 1) + row * N
            jb = g // 8
            off = (g % 8) * 16
            gidx_v[16 + jb, pl.ds(off, 16)] = gi
            for ch in range(4):
                gidx_v[jb * 4 + ch, pl.ds(off, 16)] = gi + ch * (B * N)
            return 0
        lax.fori_loop(0, KPAD // 16, mk_idx, 0, unroll=4)
        copies = []
        for jb in range(4):
            for ch in range(4):
                copies.append(pltpu.async_copy(
                    bx_hbm.at[gidx_v.at[jb * 4 + ch]],
                    gbx_v.at[pl.ds(ch * KPAD + jb * 128, 128)], sem))
            copies.append(pltpu.async_copy(
                lgf_hbm.at[gidx_v.at[16 + jb]],
                vals_v.at[pl.ds(jb * 128, 128)], sem))
        for cp in copies:
            cp.wait()

        def sigp(g, _):
            x = vals_v[pl.ds(g * 16, 16)]
            vals_v[pl.ds(g * 16, 16)] = 1.0 / (1.0 + jnp.exp(-x))
            return 0
        lax.fori_loop(0, KPAD // 16, sigp, 0, unroll=4)
        pltpu.sync_copy(vals_v, sh_vals.at[slot])
        pltpu.sync_copy(sc_hbm.at[pl.ds(row * 16, 16)], scale_v)
        svec = scale_v[...]
        sw = svec[0]
        sh_ = svec[1]

        def xform(g, _):
            cxv = gbx_v[pl.ds(g * 16, 16)]
            cyv = gbx_v[pl.ds(KPAD + g * 16, 16)]
            wv = gbx_v[pl.ds(2 * KPAD + g * 16, 16)]
            hv = gbx_v[pl.ds(3 * KPAD + g * 16, 16)]
            gbx_v[pl.ds(g * 16, 16)] = (cxv - 0.5 * wv) * sw
            gbx_v[pl.ds(KPAD + g * 16, 16)] = (cyv - 0.5 * hv) * sh_
            gbx_v[pl.ds(2 * KPAD + g * 16, 16)] = (cxv + 0.5 * wv) * sw
            gbx_v[pl.ds(3 * KPAD + g * 16, 16)] = (cyv + 0.5 * hv) * sh_
            return 0
        lax.fori_loop(0, KPAD // 16, xform, 0, unroll=4)
        pltpu.sync_copy(gbx_v, sh_bx.at[slot])

    @pl.when(h == 0)
    def _():
        def sinit(i, _):
            sk_v[pl.ds(i * 16, 16)] = cand_k[pl.ds(i * 16, 16)]
            sp_v[pl.ds(i * 16, 16)] = i * 16 + iota
            return 0
        lax.fori_loop(0, KPAD // 16, sinit, 0, unroll=4)

        size = 2
        while size <= KPAD:
            stride = size // 2
            while stride >= 1:
                if stride >= 16:
                    sv = stride // 16

                    def vstage(p, _, sv=sv, stride=stride, size=size):
                        base_a = ((p // sv) * 2 * sv + p % sv) * 16
                        base_b = base_a + stride
                        ak = sk_v[pl.ds(base_a, 16)]
                        bk = sk_v[pl.ds(base_b, 16)]
                        ap = sp_v[pl.ds(base_a, 16)]
                        bp = sp_v[pl.ds(base_b, 16)]
                        dirbit = (base_a & size) != 0
                        w = jnp.logical_or(
                            ak > bk, jnp.logical_and(ak == bk, ap < bp))
                        sw_ = jnp.where(dirbit, w, jnp.logical_not(w))
                        sk_v[pl.ds(base_a, 16)] = jnp.where(sw_, bk, ak)
                        sk_v[pl.ds(base_b, 16)] = jnp.where(sw_, ak, bk)
                        sp_v[pl.ds(base_a, 16)] = jnp.where(sw_, bp, ap)
                        sp_v[pl.ds(base_b, 16)] = jnp.where(sw_, ap, bp)
                        return 0
                    lax.fori_loop(0, KPAD // 32, vstage, 0)
                else:

                    def lstage(v, _, stride=stride, size=size):
                        base = v * 16
                        ak = sk_v[pl.ds(base, 16)]
                        ap = sp_v[pl.ds(base, 16)]
                        pidx = base + (iota ^ stride)
                        bk = plsc.load_gather(sk_v, [pidx])
                        bp = plsc.load_gather(sp_v, [pidx])
                        islower = (iota & stride) == 0
                        dirv = ((base + iota) & size) != 0
                        want_w = jnp.logical_xor(islower, dirv)
                        w = jnp.logical_or(
                            ak > bk, jnp.logical_and(ak == bk, ap < bp))
                        keep = w == want_w
                        sk_v[pl.ds(base, 16)] = jnp.where(keep, ak, bk)
                        sp_v[pl.ds(base, 16)] = jnp.where(keep, ap, bp)
                        return 0
                    lax.fori_loop(0, KPAD // 16, lstage, 0)
                stride //= 2
            size *= 2

    plsc.subcore_barrier()

    @pl.when(h == 0)
    def _():
        pltpu.sync_copy(sh_bx.at[slot], gbx_v)
        pltpu.sync_copy(sh_vals.at[slot], vals_v)

        def bperm(g, _):
            p = sp_v[pl.ds(g * 16, 16)]
            out_base = (g * 16 + iota) * 4
            for ch in range(4):
                v = plsc.load_gather(gbx_v, [ch * KPAD + p])
                plsc.store_scatter(obx_v, [out_base + ch], v)
            vov_v[pl.ds(g * 16, 16)] = plsc.load_gather(vals_v, [p])
            return 0
        lax.fori_loop(0, KPAD // 16, bperm, 0, unroll=4)
        pltpu.sync_copy(obx_v, boxes_hbm.at[pl.ds(row * KPAD * 4, KPAD * 4)])
        pltpu.sync_copy(vov_v, vals_hbm.at[pl.ds(row * KPAD, KPAD)])

    @pl.when(s == 0)
    def _():
        acc = jnp.zeros((16,), jnp.int32)
        for j in range(8):
            pltpu.sync_copy(sh_cnt.at[j], stage_v)
            cj = stage_v[...][0]
            acc = jnp.where(iota == j, cj, acc)
        cnt_v[...] = acc
        pltpu.sync_copy(cnt_v.at[pl.ds(0, 8)], cnt_hbm.at[pl.ds(c * 8, 8)])


@functools.cache
def _mk_run():
    mesh = plsc.VectorSubcoreMesh(core_axis_name="c", subcore_axis_name="s")
    return pl.kernel(
        _body,
        out_type=[
            jax.ShapeDtypeStruct((B * KPAD,), jnp.float32),
            jax.ShapeDtypeStruct((B * KPAD * 4,), jnp.float32),
            jax.ShapeDtypeStruct((B,), jnp.int32),
        ],
        mesh=mesh,
        scratch_types=[
            pltpu.VMEM((M,), jnp.int32),
            pltpu.VMEM((4096,), jnp.int32),
            pltpu.VMEM((256,), jnp.int32),
            pltpu.VMEM((256,), jnp.int32),
            pltpu.VMEM((KPAD,), jnp.int32),
            pltpu.VMEM((KPAD,), jnp.int32),
            pltpu.VMEM((KPAD,), jnp.int32),
            pltpu.VMEM((KPAD,), jnp.int32),
            pltpu.VMEM((KPAD * 4,), jnp.float32),
            pltpu.VMEM((KPAD * 4,), jnp.float32),
            pltpu.VMEM((KPAD,), jnp.float32),
            pltpu.VMEM((KPAD,), jnp.float32),
            pltpu.VMEM((20, 128), jnp.int32),
            pltpu.VMEM((16,), jnp.int32),
            pltpu.VMEM((16,), jnp.float32),
            pltpu.VMEM((16,), jnp.int32),
            pltpu.VMEM_SHARED((16, 256), jnp.int32),
            pltpu.VMEM_SHARED((16, 16), jnp.int32),
            pltpu.VMEM_SHARED((8 * 1024,), jnp.int32),
            pltpu.VMEM_SHARED((8 * 1024,), jnp.int32),
            pltpu.VMEM_SHARED((8, KPAD * 4), jnp.float32),
            pltpu.VMEM_SHARED((8, KPAD), jnp.float32),
            pltpu.VMEM_SHARED((8, 16), jnp.int32),
            pltpu.SemaphoreType.DMA,
        ],
        compiler_params=pltpu.CompilerParams(needs_layout_passes=False,
                                             use_tc_tiling_on_sc=False),
    )


@functools.partial(jax.jit, static_argnums=())
def kernel(pred_logits, pred_boxes, target_sizes):
    logits = pred_logits[..., 0].reshape(-1)
    logits_i = lax.bitcast_convert_type(logits, jnp.int32)
    boxes1d = pred_boxes.transpose(2, 0, 1).reshape(-1)
    img_h = target_sizes[:, 0].astype(jnp.float32)
    img_w = target_sizes[:, 1].astype(jnp.float32)
    scale = jnp.zeros((B, 16), jnp.float32)
    scale = scale.at[:, 0].set(img_w).at[:, 1].set(img_h).reshape(-1)

    vals_p, boxes_p, counts = _mk_run()(logits_i, logits, boxes1d, scale)
    topk_values = vals_p.reshape(B, KPAD)[:, :K]
    boxes_out = boxes_p.reshape(B, KPAD, 4)[:, :K]
    labels = jnp.zeros((B, K), dtype=jnp.int32)
    return topk_values, labels, boxes_out, counts


# speedup vs baseline: 1.1119x; 1.1119x over previous
import functools

import jax
import jax.numpy as jnp
from jax import lax
from jax.experimental import pallas as pl
from jax.experimental.pallas import tpu as pltpu
from jax.experimental.pallas import tpu_sc as plsc

B = 16
N = 20000
K = 500
M = N // 2
NV = M // 16
KPAD = 512
PAD_KEY = -(2 ** 31)
PAD_IDX = 0x7FFFFFFF
MASK31 = 0x7FFFFFFF


def _body(lgi_hbm, lgf_hbm, bx_hbm, sc_hbm,
          vals_hbm, boxes_hbm, cnt_hbm,
          keys_v, hist_v, tot_v, par_v,
          cand_k, cand_i, sk_v, sp_v,
          gbx_v, obx_v, vals_v, vov_v, gidx_v,
          stage_v, scale_v, cnt_v,
          sh_hist, sh_meta, sh_ck, sh_ci, sh_bx, sh_vals, sh_cnt,
          sem):
    c = lax.axis_index("c")
    s = lax.axis_index("s")
    slot = s // 2
    h = s % 2
    row = c * 8 + slot
    iota = lax.iota(jnp.int32, 16)
    ones16 = jnp.ones((16,), jnp.int32)
    zero16 = jnp.zeros((16,), jnp.int32)
    lane_off = iota

    pltpu.sync_copy(lgi_hbm.at[pl.ds(row * N + h * M, M)], keys_v)

    def zero_hist(i, _):
        hist_v[pl.ds(i * 16, 16)] = zero16
        return 0
    lax.fori_loop(0, 256, zero_hist, 0, unroll=8)

    def key_pass(i, cnt0):
        b = keys_v[pl.ds(i * 16, 16)]
        key = b ^ ((b >> 31) & MASK31)
        keys_v[pl.ds(i * 16, 16)] = key
        d0 = (key >> 24) + 128
        plsc.addupdate_scatter(hist_v, [d0 * 16 + lane_off], ones16)
        return cnt0 + jnp.sum((key > 0).astype(jnp.int32))
    with jax.named_scope("keypass"):
        cnt0 = lax.fori_loop(0, NV, key_pass, jnp.int32(0), unroll=5)

    q = jnp.int32(K)
    P = jnp.int32(0)
    D3 = jnp.int32(0)
    for lvl in range(4):
        if lvl > 0:
            lax.fori_loop(0, 256, zero_hist, 0, unroll=8)
            sh_amt = 24 - 8 * lvl

            def hist_pass(i, _, sh_amt=sh_amt, P=P):
                key = keys_v[pl.ds(i * 16, 16)]
                m = (key >> (sh_amt + 8)) == P
                d = (key >> sh_amt) & 0xFF
                plsc.addupdate_scatter(hist_v, [d * 16 + lane_off], ones16,
                                       mask=m)
                return 0
            with jax.named_scope("histpass"):
                lax.fori_loop(0, NV, hist_pass, 0, unroll=5)

        def lane_reduce(g, _):
            acc = zero16
            for j in range(16):
                v = hist_v[pl.ds((g * 16 + j) * 16, 16)]
                acc = jnp.where(iota == j, jnp.sum(v), acc)
            tot_v[pl.ds(g * 16, 16)] = acc
            return 0
        with jax.named_scope("lanereduce"):
            lax.fori_loop(0, 16, lane_reduce, 0, unroll=2)

        pltpu.sync_copy(tot_v, sh_hist.at[slot * 2 + h])
        plsc.subcore_barrier()
        pltpu.sync_copy(sh_hist.at[slot * 2 + (1 - h)], par_v)

        def combine(g, _):
            bse = g * 16
            par_v[pl.ds(bse, 16)] = (par_v[pl.ds(bse, 16)]
                                     + tot_v[pl.ds(bse, 16)])
            return 0
        lax.fori_loop(0, 16, combine, 0, unroll=4)

        def find_bin(j, carry, q=q):
            found, D, qn, running = carry
            g = 15 - j
            v = par_v[pl.ds(g * 16, 16)]
            ssum = jnp.sum(v)
            rev = lax.rev(v, (0,))
            cum = jnp.cumsum(rev)
            here = jnp.logical_and(jnp.logical_not(found),
                                   (running + ssum) >= q)
            crossed = (running + cum) >= q
            jstar = jnp.min(jnp.where(crossed, iota, 16))
            cum_j = jnp.sum(jnp.where(iota == jstar, cum, 0))
            rev_j = jnp.sum(jnp.where(iota == jstar, rev, 0))
            D_new = g * 16 + 15 - jstar
            q_new = q - (running + cum_j - rev_j)
            D = jnp.where(here, D_new, D)
            qn = jnp.where(here, q_new, qn)
            found = jnp.logical_or(found, here)
            return (found, D, qn, running + ssum)
        _, D, q, _ = lax.fori_loop(
            0, 16, find_bin,
            (jnp.bool_(False), jnp.int32(0), jnp.int32(0), jnp.int32(0)))
        if lvl == 0:
            P = D - 128
        else:
            P = P * 256 + D
        if lvl == 3:
            D3 = D

    T = P
    r = q
    pltpu.sync_copy(sh_hist.at[slot * 2], tot_v)
    d3base = D3 // 16 * 16
    e0vec = tot_v[pl.ds(d3base, 16)]
    e0 = jnp.sum(jnp.where(iota == D3 - d3base, e0vec, 0))
    r0 = jnp.minimum(r, e0)
    quota = jnp.where(h == 0, r0, r - r0)

    def prefill(i, _):
        cand_k[pl.ds(i * 16, 16)] = jnp.full((16,), PAD_KEY, jnp.int32)
        cand_i[pl.ds(i * 16, 16)] = jnp.full((16,), PAD_IDX, jnp.int32)
        return 0
    lax.fori_loop(0, KPAD // 16, prefill, 0, unroll=8)

    base_idx = h * M

    def compact(i, carry):
        nsel, eqrun = carry
        key = keys_v[pl.ds(i * 16, 16)]
        m_gt = key > T
        m_eq = key == T
        eq_c = jnp.cumsum(m_eq.astype(jnp.int32))
        take = jnp.logical_and(m_eq, (eqrun + eq_c) <= quota)
        m = jnp.logical_or(m_gt, take)
        mi = m.astype(jnp.int32)
        pos = jnp.cumsum(mi) - mi + nsel
        plsc.store_scatter(cand_k, [pos], key, mask=m)
        idxv = base_idx + i * 16 + iota
        plsc.store_scatter(cand_i, [pos], idxv, mask=m)
        return (nsel + jnp.sum(mi), eqrun + jnp.sum(m_eq.astype(jnp.int32)))
    with jax.named_scope("compact"):
        nsel, _ = lax.fori_loop(0, NV, compact, (jnp.int32(0), jnp.int32(0)), unroll=5)

    meta = jnp.where(iota == 0, nsel, jnp.where(iota == 1, cnt0, 0))
    stage_v[...] = meta
    pltpu.sync_copy(stage_v, sh_meta.at[slot * 2 + h])

    @pl.when(h == 0)
    def _():
        pltpu.sync_copy(cand_k, sh_ck.at[pl.ds(slot * 1024, KPAD)])
        pltpu.sync_copy(cand_i, sh_ci.at[pl.ds(slot * 1024, KPAD)])
    plsc.subcore_barrier()

    pltpu.sync_copy(sh_meta.at[slot * 2 + (1 - h)], stage_v)
    pmeta = stage_v[...]
    partner_nsel = pmeta[0]
    partner_cnt0 = pmeta[1]

    @pl.when(h == 1)
    def _():
        n0p = (partner_nsel + 7) // 8 * 8
        pltpu.sync_copy(cand_k, sh_ck.at[pl.ds(slot * 1024 + n0p, KPAD)])
        pltpu.sync_copy(cand_i, sh_ci.at[pl.ds(slot * 1024 + n0p, KPAD)])

    @pl.when(h == 0)
    def _():
        cvec = jnp.where(iota == 0, cnt0 + partner_cnt0, 0)
        cnt_v[...] = cvec
        pltpu.sync_copy(cnt_v, sh_cnt.at[slot])
    plsc.subcore_barrier()

    pltpu.sync_copy(sh_ck.at[pl.ds(slot * 1024, KPAD)], cand_k)
    pltpu.sync_copy(sh_ci.at[pl.ds(slot * 1024, KPAD)], cand_i)

    @pl.when(h == 1)
    def _():
        def mk_idx(g, _):
            ci = cand_i[pl.ds(g * 16, 16)]
            gi = jnp.minimum(ci, N - 1) + row * N
            jb = g // 8
            off = (g % 8) * 16
            gidx_v[16 + jb, pl.ds(off, 16)] = gi
            for ch in range(4):
                gidx_v[jb * 4 + ch, pl.ds(off, 16)] = gi + ch * (B * N)
            return 0
        lax.fori_loop(0, KPAD // 16, mk_idx, 0, unroll=4)
        copies = []
        for jb in range(4):
            for ch in range(4):
                copies.append(pltpu.async_copy(
                    bx_hbm.at[gidx_v.at[jb * 4 + ch]],
                    gbx_v.at[pl.ds(ch * KPAD + jb * 128, 128)], sem))
            copies.append(pltpu.async_copy(
                lgf_hbm.at[gidx_v.at[16 + jb]],
                vals_v.at[pl.ds(jb * 128, 128)], sem))
        for cp in copies:
            cp.wait()

        def sigp(g, _):
            x = vals_v[pl.ds(g * 16, 16)]
            vals_v[pl.ds(g * 16, 16)] = 1.0 / (1.0 + jnp.exp(-x))
            return 0
        lax.fori_loop(0, KPAD // 16, sigp, 0, unroll=4)
        pltpu.sync_copy(vals_v, sh_vals.at[slot])
        pltpu.sync_copy(sc_hbm.at[pl.ds(row * 16, 16)], scale_v)
        svec = scale_v[...]
        sw = svec[0]
        sh_ = svec[1]

        def xform(g, _):
            cxv = gbx_v[pl.ds(g * 16, 16)]
            cyv = gbx_v[pl.ds(KPAD + g * 16, 16)]
            wv = gbx_v[pl.ds(2 * KPAD + g * 16, 16)]
            hv = gbx_v[pl.ds(3 * KPAD + g * 16, 16)]
            gbx_v[pl.ds(g * 16, 16)] = (cxv - 0.5 * wv) * sw
            gbx_v[pl.ds(KPAD + g * 16, 16)] = (cyv - 0.5 * hv) * sh_
            gbx_v[pl.ds(2 * KPAD + g * 16, 16)] = (cxv + 0.5 * wv) * sw
            gbx_v[pl.ds(3 * KPAD + g * 16, 16)] = (cyv + 0.5 * hv) * sh_
            return 0
        lax.fori_loop(0, KPAD // 16, xform, 0, unroll=4)
        pltpu.sync_copy(gbx_v, sh_bx.at[slot])

    @pl.when(h == 0)
    def _():
        def sinit(i, _):
            sk_v[pl.ds(i * 16, 16)] = cand_k[pl.ds(i * 16, 16)]
            sp_v[pl.ds(i * 16, 16)] = i * 16 + iota
            return 0
        lax.fori_loop(0, KPAD // 16, sinit, 0, unroll=4)

        size = 2
        while size <= KPAD:
            stride = size // 2
            while stride >= 1:
                if stride >= 16:
                    sv = stride // 16

                    def vstage(p, _, sv=sv, stride=stride, size=size):
                        base_a = ((p // sv) * 2 * sv + p % sv) * 16
                        base_b = base_a + stride
                        ak = sk_v[pl.ds(base_a, 16)]
                        bk = sk_v[pl.ds(base_b, 16)]
                        ap = sp_v[pl.ds(base_a, 16)]
                        bp = sp_v[pl.ds(base_b, 16)]
                        dirbit = (base_a & size) != 0
                        w = jnp.logical_or(
                            ak > bk, jnp.logical_and(ak == bk, ap < bp))
                        sw_ = jnp.where(dirbit, w, jnp.logical_not(w))
                        sk_v[pl.ds(base_a, 16)] = jnp.where(sw_, bk, ak)
                        sk_v[pl.ds(base_b, 16)] = jnp.where(sw_, ak, bk)
                        sp_v[pl.ds(base_a, 16)] = jnp.where(sw_, bp, ap)
                        sp_v[pl.ds(base_b, 16)] = jnp.where(sw_, ap, bp)
                        return 0
                    lax.fori_loop(0, KPAD // 32, vstage, 0)
                else:

                    def lstage(v, _, stride=stride, size=size):
                        base = v * 16
                        ak = sk_v[pl.ds(base, 16)]
                        ap = sp_v[pl.ds(base, 16)]
                        pidx = base + (iota ^ stride)
                        bk = plsc.load_gather(sk_v, [pidx])
                        bp = plsc.load_gather(sp_v, [pidx])
                        islower = (iota & stride) == 0
                        dirv = ((base + iota) & size) != 0
                        want_w = jnp.logical_xor(islower, dirv)
                        w = jnp.logical_or(
                            ak > bk, jnp.logical_and(ak == bk, ap < bp))
                        keep = w == want_w
                        sk_v[pl.ds(base, 16)] = jnp.where(keep, ak, bk)
                        sp_v[pl.ds(base, 16)] = jnp.where(keep, ap, bp)
                        return 0
                    lax.fori_loop(0, KPAD // 16, lstage, 0)
                stride //= 2
            size *= 2

    plsc.subcore_barrier()

    @pl.when(h == 0)
    def _():
        pltpu.sync_copy(sh_bx.at[slot], gbx_v)
        pltpu.sync_copy(sh_vals.at[slot], vals_v)

        def bperm(g, _):
            p = sp_v[pl.ds(g * 16, 16)]
            out_base = (g * 16 + iota) * 4
            for ch in range(4):
                v = plsc.load_gather(gbx_v, [ch * KPAD + p])
                plsc.store_scatter(obx_v, [out_base + ch], v)
            vov_v[pl.ds(g * 16, 16)] = plsc.load_gather(vals_v, [p])
            return 0
        lax.fori_loop(0, KPAD // 16, bperm, 0, unroll=4)
        pltpu.sync_copy(obx_v, boxes_hbm.at[pl.ds(row * KPAD * 4, KPAD * 4)])
        pltpu.sync_copy(vov_v, vals_hbm.at[pl.ds(row * KPAD, KPAD)])

    @pl.when(s == 0)
    def _():
        acc = jnp.zeros((16,), jnp.int32)
        for j in range(8):
            pltpu.sync_copy(sh_cnt.at[j], stage_v)
            cj = stage_v[...][0]
            acc = jnp.where(iota == j, cj, acc)
        cnt_v[...] = acc
        pltpu.sync_copy(cnt_v.at[pl.ds(0, 8)], cnt_hbm.at[pl.ds(c * 8, 8)])


@functools.cache
def _mk_run():
    mesh = plsc.VectorSubcoreMesh(core_axis_name="c", subcore_axis_name="s")
    return pl.kernel(
        _body,
        out_type=[
            jax.ShapeDtypeStruct((B * KPAD,), jnp.float32),
            jax.ShapeDtypeStruct((B * KPAD * 4,), jnp.float32),
            jax.ShapeDtypeStruct((B,), jnp.int32),
        ],
        mesh=mesh,
        scratch_types=[
            pltpu.VMEM((M,), jnp.int32),
            pltpu.VMEM((4096,), jnp.int32),
            pltpu.VMEM((256,), jnp.int32),
            pltpu.VMEM((256,), jnp.int32),
            pltpu.VMEM((KPAD,), jnp.int32),
            pltpu.VMEM((KPAD,), jnp.int32),
            pltpu.VMEM((KPAD,), jnp.int32),
            pltpu.VMEM((KPAD,), jnp.int32),
            pltpu.VMEM((KPAD * 4,), jnp.float32),
            pltpu.VMEM((KPAD * 4,), jnp.float32),
            pltpu.VMEM((KPAD,), jnp.float32),
            pltpu.VMEM((KPAD,), jnp.float32),
            pltpu.VMEM((20, 128), jnp.int32),
            pltpu.VMEM((16,), jnp.int32),
            pltpu.VMEM((16,), jnp.float32),
            pltpu.VMEM((16,), jnp.int32),
            pltpu.VMEM_SHARED((16, 256), jnp.int32),
            pltpu.VMEM_SHARED((16, 16), jnp.int32),
            pltpu.VMEM_SHARED((8 * 1024,), jnp.int32),
            pltpu.VMEM_SHARED((8 * 1024,), jnp.int32),
            pltpu.VMEM_SHARED((8, KPAD * 4), jnp.float32),
            pltpu.VMEM_SHARED((8, KPAD), jnp.float32),
            pltpu.VMEM_SHARED((8, 16), jnp.int32),
            pltpu.SemaphoreType.DMA,
        ],
        compiler_params=pltpu.CompilerParams(needs_layout_passes=False,
                                             use_tc_tiling_on_sc=False),
    )


@functools.partial(jax.jit, static_argnums=())
def kernel(pred_logits, pred_boxes, target_sizes):
    logits = pred_logits[..., 0].reshape(-1)
    logits_i = lax.bitcast_convert_type(logits, jnp.int32)
    boxes1d = pred_boxes.transpose(2, 0, 1).reshape(-1)
    img_h = target_sizes[:, 0].astype(jnp.float32)
    img_w = target_sizes[:, 1].astype(jnp.float32)
    scale = jnp.zeros((B, 16), jnp.float32)
    scale = scale.at[:, 0].set(img_w).at[:, 1].set(img_h).reshape(-1)

    vals_p, boxes_p, counts = _mk_run()(logits_i, logits, boxes1d, scale)
    topk_values = vals_p.reshape(B, KPAD)[:, :K]
    boxes_out = boxes_p.reshape(B, KPAD, 4)[:, :K]
    labels = jnp.zeros((B, K), dtype=jnp.int32)
    return topk_values, labels, boxes_out, counts
